# baseline (speedup 1.0000x reference)
import jax
import jax.numpy as jnp
from jax.experimental import pallas as pl
from jax.experimental.pallas import tpu as pltpu

_SUBLANE = 8
_NC = 2


def _round_up(n, m):
    return ((n + m - 1) // m) * m


def _leaky(v, slope=0.01):
    return jnp.where(v > 0, v, slope * v)


def _mlp_body(x_ref, w1_ref, b1_ref, w2_ref, b2_ref, o_ref):
    xb = x_ref[...]
    hid = w1_ref.shape[1]
    ck = hid // _NC
    acc = None
    for c in range(_NC):
        sl = slice(c * ck, (c + 1) * ck)
        h = jnp.dot(xb, w1_ref[:, sl], preferred_element_type=jnp.float32)
        h = _leaky(h + b1_ref[:, sl])
        p = jnp.dot(h, w2_ref[sl, :], preferred_element_type=jnp.float32)
        acc = p if acc is None else acc + p
    y = _leaky(acc + b2_ref[...])
    o_ref[...] = y.astype(o_ref.dtype)


def kernel(x, w1, b1, w2, b2, *, tm=512):
    B, in_size = x.shape
    hid = w1.shape[1]
    out_size = w2.shape[1]
    dt = x.dtype

    b1 = b1.reshape(1, hid)
    b2 = b2.reshape(1, out_size)

    b_p = _round_up(B, _SUBLANE)
    xp = x if b_p == B else jnp.zeros((b_p, in_size), dt).at[:B].set(x)

    tm_eff = min(tm, max(_SUBLANE, _round_up(pl.cdiv(b_p, 2), _SUBLANE)))
    grid = (pl.cdiv(b_p, tm_eff),)

    cost = pl.CostEstimate(
        flops=2 * b_p * (in_size * hid + hid * out_size),
        transcendentals=0,
        bytes_accessed=(b_p * in_size + in_size * hid + hid
                        + hid * out_size + out_size + b_p * out_size) * 4,
    )

    out = pl.pallas_call(
        _mlp_body,
        out_shape=jax.ShapeDtypeStruct((b_p, out_size), dt),
        grid_spec=pltpu.PrefetchScalarGridSpec(
            num_scalar_prefetch=0,
            grid=grid,
            in_specs=[
                pl.BlockSpec((tm_eff, in_size), lambda i: (i, 0)),
                pl.BlockSpec((in_size, hid), lambda i: (0, 0)),
                pl.BlockSpec((1, hid), lambda i: (0, 0)),
                pl.BlockSpec((hid, out_size), lambda i: (0, 0)),
                pl.BlockSpec((1, out_size), lambda i: (0, 0)),
            ],
            out_specs=pl.BlockSpec((tm_eff, out_size), lambda i: (i, 0)),
        ),
        compiler_params=pltpu.CompilerParams(
            dimension_semantics=("parallel",),
        ),
        cost_estimate=cost,
    )(xp, w1, b1, w2, b2)

    return out if b_p == B else out[:B]
